# baseline (speedup 1.0000x reference)
import jax
import jax.numpy as jnp
from jax import lax
from jax.experimental import pallas as pl
from jax.experimental.pallas import tpu as pltpu, tpu_sc as plsc

_NE = 100_000
_NRB = 1_600
_NF = 3_200_000
_D = 16
_NREL = 200
_NC = 2
_NS = 16
_NW = _NC * _NS
_CHUNK = 1024
_SUB = 128
_NSUB = _CHUNK // _SUB
_NCHUNKS = _NF // _CHUNK
_IDXROWS = _NF // _SUB
_EROWS_PER_TILE = _NE // _NS
_RROWS_PER_TILE = _NRB // _NS
_OUT_ROWS = 2 * _NE + _NRB + _NF
_FACT_OFF = 2 * _NE + _NRB
_HROWS_PER_W = _NE // _NW
_RROWS_PER_W = _NRB // _NW


def _zero_rows(buf, n):
    z = jnp.zeros((_D,), jnp.float32)

    def body(r, carry):
        buf[r] = z
        return carry

    lax.fori_loop(0, n, body, 0)


_ECHUNK = 1000
_NECHUNK = _NE // _ECHUNK
_RCHUNK = 200
_NRCHUNK = _NRB // _RCHUNK


def _p1_body(heads1, tails1, rels1, ids1, w1, val2,
             head_out, tail_out, rel_out,
             acc_sh, rel_sh, val_v, w_v, sidx_v, rels_v, ids_v,
             ridx_v, sem):
    c = lax.axis_index("c")
    s = lax.axis_index("s")

    _zero_rows(val_v, _CHUNK)
    nz = (_NECHUNK // _NS) + jnp.where(s < _NECHUNK % _NS, 1, 0)

    def zbody(i, carry):
        pltpu.sync_copy(val_v.at[pl.ds(0, _ECHUNK)],
                        acc_sh.at[pl.ds((s + i * _NS) * _ECHUNK, _ECHUNK)])
        return carry

    lax.fori_loop(0, nz, zbody, 0)

    @pl.when(s < _NRCHUNK)
    def _():
        pltpu.sync_copy(val_v.at[pl.ds(0, _RCHUNK)],
                        rel_sh.at[pl.ds(s * _RCHUNK, _RCHUNK)])

    plsc.subcore_barrier()

    nloc = (_NCHUNKS // _NS) + jnp.where(s < (_NCHUNKS % _NS), 1, 0)

    def make_chunk_body(do_rel):
        def chunk_body(i, carry):
            cid = s + i * _NS
            base = cid * _CHUNK
            idx_src = heads1 if do_rel else tails1
            cps = [
                pltpu.make_async_copy(val2.at[pl.ds(base, _CHUNK)], val_v, sem),
                pltpu.make_async_copy(w1.at[pl.ds(base, _CHUNK)], w_v, sem),
            ]
            cps += [
                pltpu.make_async_copy(
                    idx_src.at[pl.ds(base + j * _SUB, _SUB)],
                    sidx_v.at[j], sem)
                for j in range(_NSUB)
            ]
            if do_rel:
                cps.append(pltpu.make_async_copy(
                    rels1.at[pl.ds(base, _CHUNK)], rels_v, sem))
                cps.append(pltpu.make_async_copy(
                    ids1.at[pl.ds(base, _CHUNK)], ids_v, sem))
            for cp in cps:
                cp.start()
            for cp in cps:
                cp.wait()

            def groupfn(g, rcarry):
                gb = g * 16
                w16 = w_v[pl.ds(gb, 16)]
                if do_rel:
                    j = g // 8
                    col = (g % 8) * 16
                    ridx_v[j, pl.ds(col, 16)] = (
                        rels_v[pl.ds(gb, 16)] + ids_v[pl.ds(gb, 16)] * _NREL)
                for r in range(16):
                    val_v[gb + r] = val_v[gb + r] * w16[r]
                return rcarry

            lax.fori_loop(0, _CHUNK // 16, groupfn, 0)

            scs = [pltpu.make_async_copy(val_v.at[pl.ds(j * _SUB, _SUB)],
                                         acc_sh.at[sidx_v.at[j]], sem)
                   for j in range(_NSUB)]
            if do_rel:
                scs += [pltpu.make_async_copy(val_v.at[pl.ds(j * _SUB, _SUB)],
                                              rel_sh.at[ridx_v.at[j]], sem)
                        for j in range(_NSUB)]
            for sc_ in scs:
                sc_.start(add=True)
            for sc_ in scs:
                sc_.wait()
            return carry

        return chunk_body

    @pl.when(c == 0)
    def _():
        lax.fori_loop(0, nloc, make_chunk_body(True), 0)

    @pl.when(c != 0)
    def _():
        lax.fori_loop(0, nloc, make_chunk_body(False), 0)

    plsc.subcore_barrier()

    def make_wb(dst):
        def wb(i, carry):
            b = (s + i * _NS) * _ECHUNK
            pltpu.sync_copy(acc_sh.at[pl.ds(b, _ECHUNK)],
                            dst.at[pl.ds(b, _ECHUNK)])
            return carry

        return wb

    @pl.when(c == 0)
    def _():
        lax.fori_loop(0, nz, make_wb(head_out), 0)

        @pl.when(s < _NRCHUNK)
        def _():
            pltpu.sync_copy(rel_sh.at[pl.ds(s * _RCHUNK, _RCHUNK)],
                            rel_out.at[pl.ds(s * _RCHUNK, _RCHUNK)])

    @pl.when(c != 0)
    def _():
        lax.fori_loop(0, nz, make_wb(tail_out), 0)


def _p2_body(heads1, w1, head_in, tail_in, rel_in, out,
             val_v, w_v, gidx_v, sem):
    c = lax.axis_index("c")
    s = lax.axis_index("s")
    w = s * _NC + c

    ncp = (_NECHUNK // _NW) + jnp.where(w < _NECHUNK % _NW, 1, 0)

    def cbody(i, carry):
        b = (w + i * _NW) * _ECHUNK
        pltpu.sync_copy(head_in.at[pl.ds(b, _ECHUNK)],
                        val_v.at[pl.ds(0, _ECHUNK)])
        pltpu.sync_copy(val_v.at[pl.ds(0, _ECHUNK)],
                        out.at[pl.ds(b, _ECHUNK)])
        pltpu.sync_copy(tail_in.at[pl.ds(b, _ECHUNK)],
                        val_v.at[pl.ds(0, _ECHUNK)])
        pltpu.sync_copy(val_v.at[pl.ds(0, _ECHUNK)],
                        out.at[pl.ds(_NE + b, _ECHUNK)])
        return carry

    lax.fori_loop(0, ncp, cbody, 0)

    @pl.when(w < _NRCHUNK)
    def _():
        pltpu.sync_copy(rel_in.at[pl.ds(w * _RCHUNK, _RCHUNK)],
                        val_v.at[pl.ds(0, _RCHUNK)])
        pltpu.sync_copy(val_v.at[pl.ds(0, _RCHUNK)],
                        out.at[pl.ds(2 * _NE + w * _RCHUNK, _RCHUNK)])

    nloc = (_NCHUNKS // _NW) + jnp.where(w < (_NCHUNKS % _NW), 1, 0)

    def chunk_body(i, carry):
        cid = w + i * _NW
        base = cid * _CHUNK
        cps = [
            pltpu.make_async_copy(w1.at[pl.ds(base, _CHUNK)], w_v, sem),
        ]
        cps += [
            pltpu.make_async_copy(heads1.at[pl.ds(base + j * _SUB, _SUB)],
                                  gidx_v.at[j], sem)
            for j in range(_NSUB)
        ]
        for cp in cps:
            cp.start()
        for cp in cps:
            cp.wait()

        gs = [pltpu.make_async_copy(head_in.at[gidx_v.at[j]],
                                    val_v.at[pl.ds(j * _SUB, _SUB)], sem)
              for j in range(_NSUB)]
        for g in gs:
            g.start()
        for g in gs:
            g.wait()

        def groupfn(g, rcarry):
            gb = g * 16
            w16 = w_v[pl.ds(gb, 16)]
            for r in range(16):
                val_v[gb + r] = val_v[gb + r] * w16[r]
            return rcarry

        lax.fori_loop(0, _CHUNK // 16, groupfn, 0)
        pltpu.sync_copy(val_v, out.at[pl.ds(_FACT_OFF + base, _CHUNK)])
        return carry

    lax.fori_loop(0, nloc, chunk_body, 0)


_RROWS = _OUT_ROWS // 8
_RBLK = 400


def _repack_body(x_ref, o_ref):
    for k in range(8):
        o_ref[:, k, :] = x_ref[:, pl.ds(k * _D, _D)]


def kernel(batch_heads, batch_rels, batch_tails, batch_ids, fact_ids,
           weight_list, fact_val):
    del fact_ids
    mesh = plsc.VectorSubcoreMesh(core_axis_name="c", subcore_axis_name="s")

    f32 = jnp.float32
    cparams = pltpu.CompilerParams(use_tc_tiling_on_sc=False)
    p1 = pl.kernel(
        _p1_body,
        out_type=(
            jax.ShapeDtypeStruct((_NE, _D), f32),
            jax.ShapeDtypeStruct((_NE, _D), f32),
            jax.ShapeDtypeStruct((_NRB, _D), f32),
        ),
        mesh=mesh,
        scratch_types=[
            pltpu.VMEM_SHARED((_NE, _D), f32),
            pltpu.VMEM_SHARED((_NRB, _D), f32),
            pltpu.VMEM((_CHUNK, _D), f32),
            pltpu.VMEM((_CHUNK,), f32),
            pltpu.VMEM((_NSUB, _SUB), jnp.int32),
            pltpu.VMEM((_CHUNK,), jnp.int32),
            pltpu.VMEM((_CHUNK,), jnp.int32),
            pltpu.VMEM((_NSUB, _SUB), jnp.int32),
            pltpu.SemaphoreType.DMA,
        ],
        compiler_params=cparams,
    )
    head_agg, tail_agg, rel_agg = p1(batch_heads, batch_tails, batch_rels,
                                     batch_ids, weight_list, fact_val)

    p2 = pl.kernel(
        _p2_body,
        out_type=jax.ShapeDtypeStruct((_OUT_ROWS, _D), f32),
        mesh=mesh,
        scratch_types=[
            pltpu.VMEM((_CHUNK, _D), f32),
            pltpu.VMEM((_CHUNK,), f32),
            pltpu.VMEM((_NSUB, _SUB), jnp.int32),
            pltpu.SemaphoreType.DMA,
        ],
        compiler_params=cparams,
    )
    lin = p2(batch_heads, weight_list, head_agg, tail_agg, rel_agg)

    repack = pl.pallas_call(
        _repack_body,
        grid=(_RROWS // _RBLK,),
        in_specs=[pl.BlockSpec((_RBLK, 128), lambda i: (i, 0))],
        out_specs=pl.BlockSpec((_RBLK, 8, _D), lambda i: (i, 0, 0)),
        out_shape=jax.ShapeDtypeStruct((_RROWS, 8, _D), f32),
    )
    return repack(lin.reshape(_RROWS, 128)).reshape(_OUT_ROWS, _D)


# speedup vs baseline: 1.2775x; 1.2775x over previous
import jax
import jax.numpy as jnp
from jax import lax
from jax.experimental import pallas as pl
from jax.experimental.pallas import tpu as pltpu, tpu_sc as plsc

_NE = 100_000
_NRB = 1_600
_NF = 3_200_000
_D = 16
_NREL = 200
_NC = 2
_NS = 16
_NW = _NC * _NS
_OUT_ROWS = 2 * _NE + _NRB + _NF
_OTC = _OUT_ROWS // 128
_VTC = _NF // 128
_HDRTC = (2 * _NE + _NRB) // 128
_HT_COL = _NE // 128
_TR_COL = 2 * _NE // 128

_C1 = 512
_NSUB1 = _C1 // 128
_NCH1 = _NF // _C1
_C2 = 1024
_NSUB2 = _C2 // 128
_NCH2 = _NF // _C2
_EZ = 400
_NEZ = _NE // _EZ


def _lane_consts(stride):
    iota = lax.iota(jnp.int32, 16)
    return (iota // 8) * stride + iota % 8, iota


def _splat(x):
    return jnp.broadcast_to(x, (16,))


def _p1_body(heads1, tails1, rels1, ids1, w1, val3,
             head_out, tail_out, rel_out,
             acc_sh, rel_sh, vv, val16_v, w_v, sidx_v, rels_v, ids_v,
             ridx_v, sem):
    c = lax.axis_index("c")
    s = lax.axis_index("s")
    b1v, _ = _lane_consts(_NSUB1 * 8)

    z = jnp.zeros((_D,), jnp.float32)

    def zb(r, carry):
        val16_v[r] = z
        return carry

    lax.fori_loop(0, _C1, zb, 0)
    nz = (_NEZ // _NS) + jnp.where(s < _NEZ % _NS, 1, 0)

    def zbody(i, carry):
        pltpu.sync_copy(val16_v.at[pl.ds(0, _EZ)],
                        acc_sh.at[pl.ds((s + i * _NS) * _EZ, _EZ)])
        return carry

    lax.fori_loop(0, nz, zbody, 0)

    @pl.when(s < _NRB // _EZ)
    def _():
        pltpu.sync_copy(val16_v.at[pl.ds(0, _EZ)],
                        rel_sh.at[pl.ds(s * _EZ, _EZ)])

    plsc.subcore_barrier()

    nloc = (_NCH1 // _NS) + jnp.where(s < _NCH1 % _NS, 1, 0)

    def make_chunk_body(do_rel):
        def chunk_body(i, carry):
            cid = s + i * _NS
            base = cid * _C1
            tc0 = cid * _NSUB1
            idx_src = heads1 if do_rel else tails1
            cps = [
                pltpu.make_async_copy(val3.at[0, pl.ds(tc0 * 8, _NSUB1 * 8)],
                                      vv.at[pl.ds(0, _NSUB1 * 8)], sem),
                pltpu.make_async_copy(val3.at[1, pl.ds(tc0 * 8, _NSUB1 * 8)],
                                      vv.at[pl.ds(_NSUB1 * 8, _NSUB1 * 8)],
                                      sem),
                pltpu.make_async_copy(w1.at[pl.ds(base, _C1)], w_v, sem),
            ]
            cps += [
                pltpu.make_async_copy(
                    idx_src.at[pl.ds(base + j * 128, 128)],
                    sidx_v.at[j], sem)
                for j in range(_NSUB1)
            ]
            if do_rel:
                cps.append(pltpu.make_async_copy(
                    rels1.at[pl.ds(base, _C1)], rels_v, sem))
                cps.append(pltpu.make_async_copy(
                    ids1.at[pl.ds(base, _C1)], ids_v, sem))
            for cp in cps:
                cp.start()
            for cp in cps:
                cp.wait()

            def groupfn(g, rcarry):
                gb = g * 16
                w16 = w_v[pl.ds(gb, 16)]
                if do_rel:
                    ridx_v[g // 8, pl.ds((g % 8) * 16, 16)] = (
                        rels_v[pl.ds(gb, 16)] + ids_v[pl.ds(gb, 16)] * _NREL)
                for r in range(16):
                    f = gb + r
                    row = plsc.load_gather(
                        vv, [b1v + (f // 128) * 8, _splat(f % 128)])
                    val16_v[f] = row * w16[r]
                return rcarry

            lax.fori_loop(0, _C1 // 16, groupfn, 0)

            scs = [pltpu.make_async_copy(val16_v.at[pl.ds(j * 128, 128)],
                                         acc_sh.at[sidx_v.at[j]], sem)
                   for j in range(_NSUB1)]
            if do_rel:
                scs += [pltpu.make_async_copy(
                    val16_v.at[pl.ds(j * 128, 128)],
                    rel_sh.at[ridx_v.at[j]], sem)
                    for j in range(_NSUB1)]
            for sc_ in scs:
                sc_.start(add=True)
            for sc_ in scs:
                sc_.wait()
            return carry

        return chunk_body

    @pl.when(c == 0)
    def _():
        lax.fori_loop(0, nloc, make_chunk_body(True), 0)

    @pl.when(c != 0)
    def _():
        lax.fori_loop(0, nloc, make_chunk_body(False), 0)

    plsc.subcore_barrier()

    def make_wb(dst):
        def wb(i, carry):
            b = (s + i * _NS) * _EZ
            pltpu.sync_copy(acc_sh.at[pl.ds(b, _EZ)], dst.at[pl.ds(b, _EZ)])
            return carry

        return wb

    @pl.when(c == 0)
    def _():
        lax.fori_loop(0, nz, make_wb(head_out), 0)

        @pl.when(s < _NRB // _EZ)
        def _():
            pltpu.sync_copy(rel_sh.at[pl.ds(s * _EZ, _EZ)],
                            rel_out.at[pl.ds(s * _EZ, _EZ)])

    @pl.when(c != 0)
    def _():
        lax.fori_loop(0, nz, make_wb(tail_out), 0)


def _p2_body(heads1, w1, head_in, tail_in, rel_in, out3,
             val_v, ov, w_v, gidx_v, sbuf, ov1, sem):
    c = lax.axis_index("c")
    s = lax.axis_index("s")
    w = s * _NC + c
    b2v, iv = _lane_consts(_NSUB2 * 8)
    b3v, _ = _lane_consts(8)

    ncp = (_HDRTC // _NW) + jnp.where(w < _HDRTC % _NW, 1, 0)

    def colfn(i, carry):
        t = w + i * _NW
        rbase = t * 128

        @pl.when(t < _HT_COL)
        def _():
            pltpu.sync_copy(head_in.at[pl.ds(rbase, 128)], sbuf)

        @pl.when(t == _HT_COL)
        def _():
            pltpu.sync_copy(head_in.at[pl.ds(_HT_COL * 128, _NE % 128)],
                            sbuf.at[pl.ds(0, _NE % 128)])
            pltpu.sync_copy(tail_in.at[pl.ds(0, 128 - _NE % 128)],
                            sbuf.at[pl.ds(_NE % 128, 128 - _NE % 128)])

        @pl.when(jnp.logical_and(t > _HT_COL, t < _TR_COL))
        def _():
            pltpu.sync_copy(tail_in.at[pl.ds(rbase - _NE, 128)], sbuf)

        @pl.when(t == _TR_COL)
        def _():
            pltpu.sync_copy(tail_in.at[pl.ds(_TR_COL * 128 - _NE, 64)],
                            sbuf.at[pl.ds(0, 64)])
            pltpu.sync_copy(rel_in.at[pl.ds(0, 64)], sbuf.at[pl.ds(64, 64)])

        @pl.when(t > _TR_COL)
        def _():
            pltpu.sync_copy(rel_in.at[pl.ds(rbase - 2 * _NE, 128)], sbuf)

        def tb(g, rcarry):
            gb = g * 16
            for r in range(16):
                f = gb + r
                plsc.store_scatter(ov1, [b3v, _splat(f)], sbuf[f])
            return rcarry

        lax.fori_loop(0, 8, tb, 0)
        pltpu.sync_copy(ov1.at[pl.ds(0, 8)], out3.at[0, pl.ds(t * 8, 8)])
        pltpu.sync_copy(ov1.at[pl.ds(8, 8)], out3.at[1, pl.ds(t * 8, 8)])
        return carry

    lax.fori_loop(0, ncp, colfn, 0)

    nloc = (_NCH2 // _NW) + jnp.where(w < _NCH2 % _NW, 1, 0)

    def chunk_body(i, carry):
        cid = w + i * _NW
        base = cid * _C2
        cps = [pltpu.make_async_copy(w1.at[pl.ds(base, _C2)], w_v, sem)]
        cps += [
            pltpu.make_async_copy(heads1.at[pl.ds(base + j * 128, 128)],
                                  gidx_v.at[j], sem)
            for j in range(_NSUB2)
        ]
        for cp in cps:
            cp.start()
        for cp in cps:
            cp.wait()

        gs = [pltpu.make_async_copy(head_in.at[gidx_v.at[j]],
                                    val_v.at[pl.ds(j * 128, 128)], sem)
              for j in range(_NSUB2)]
        for g in gs:
            g.start()
        for g in gs:
            g.wait()

        def groupfn(g, rcarry):
            gb = g * 16
            w16 = w_v[pl.ds(gb, 16)]
            for r in range(16):
                f = gb + r
                row = val_v[f] * w16[r]
                plsc.store_scatter(
                    ov, [b2v + (f // 128) * 8, _splat(f % 128)], row)
            return rcarry

        lax.fori_loop(0, _C2 // 16, groupfn, 0)
        mo = (_HDRTC + cid * _NSUB2) * 8
        nm = _NSUB2 * 8
        pltpu.sync_copy(ov.at[pl.ds(0, nm)], out3.at[0, pl.ds(mo, nm)])
        pltpu.sync_copy(ov.at[pl.ds(nm, nm)], out3.at[1, pl.ds(mo, nm)])
        return carry

    lax.fori_loop(0, nloc, chunk_body, 0)


def kernel(batch_heads, batch_rels, batch_tails, batch_ids, fact_ids,
           weight_list, fact_val):
    del fact_ids
    val3 = fact_val.reshape(_VTC, 128, 2, 8).transpose(2, 0, 3, 1).reshape(
        2, _NF // 16, 128)
    mesh = plsc.VectorSubcoreMesh(core_axis_name="c", subcore_axis_name="s")

    f32 = jnp.float32
    i32 = jnp.int32
    cparams = pltpu.CompilerParams(use_tc_tiling_on_sc=False,
                                   needs_layout_passes=False)
    p1 = pl.kernel(
        _p1_body,
        out_type=(
            jax.ShapeDtypeStruct((_NE, _D), f32),
            jax.ShapeDtypeStruct((_NE, _D), f32),
            jax.ShapeDtypeStruct((_NRB, _D), f32),
        ),
        mesh=mesh,
        scratch_types=[
            pltpu.VMEM_SHARED((_NE, _D), f32),
            pltpu.VMEM_SHARED((_NRB, _D), f32),
            pltpu.VMEM((2 * _NSUB1 * 8, 128), f32),
            pltpu.VMEM((_C1, _D), f32),
            pltpu.VMEM((_C1,), f32),
            pltpu.VMEM((_NSUB1, 128), i32),
            pltpu.VMEM((_C1,), i32),
            pltpu.VMEM((_C1,), i32),
            pltpu.VMEM((_NSUB1, 128), i32),
            pltpu.SemaphoreType.DMA,
        ],
        compiler_params=cparams,
    )
    head_agg, tail_agg, rel_agg = p1(batch_heads, batch_tails, batch_rels,
                                     batch_ids, weight_list, val3)

    p2 = pl.kernel(
        _p2_body,
        out_type=jax.ShapeDtypeStruct((2, _OTC * 8, 128), f32),
        mesh=mesh,
        scratch_types=[
            pltpu.VMEM((_C2, _D), f32),
            pltpu.VMEM((2 * _NSUB2 * 8, 128), f32),
            pltpu.VMEM((_C2,), f32),
            pltpu.VMEM((_NSUB2, 128), i32),
            pltpu.VMEM((128, _D), f32),
            pltpu.VMEM((16, 128), f32),
            pltpu.SemaphoreType.DMA,
        ],
        compiler_params=cparams,
    )
    out3 = p2(batch_heads, weight_list, head_agg, tail_agg, rel_agg)
    return out3.reshape(2, _OTC, 8, 128).transpose(1, 3, 0, 2).reshape(
        _OUT_ROWS, _D)


# speedup vs baseline: 1.2786x; 1.0009x over previous
import jax
import jax.numpy as jnp
from jax import lax
from jax.experimental import pallas as pl
from jax.experimental.pallas import tpu as pltpu, tpu_sc as plsc

_NE = 100_000
_NRB = 1_600
_NF = 3_200_000
_D = 16
_NREL = 200
_NC = 2
_NS = 16
_NW = _NC * _NS
_OUT_ROWS = 2 * _NE + _NRB + _NF
_OTC = _OUT_ROWS // 128
_VTC = _NF // 128
_HDRTC = (2 * _NE + _NRB) // 128
_HT_COL = _NE // 128
_TR_COL = 2 * _NE // 128

_C1 = 512
_NSUB1 = _C1 // 128
_NCH1 = _NF // _C1
_C2 = 1024
_NSUB2 = _C2 // 128
_NCH2 = _NF // _C2
_EZ = 400
_NEZ = _NE // _EZ


def _lane_consts(stride):
    iota = lax.iota(jnp.int32, 16)
    return (iota // 8) * stride + iota % 8, iota


def _splat(x):
    return jnp.broadcast_to(x, (16,))


def _p1_body(heads1, tails1, rels1, ids1, w1, val3,
             head_out, tail_out, rel_out,
             acc_sh, rel_sh, vv, val16_v, w_v, sidx_v, rels_v, ids_v,
             ridx_v, sem):
    c = lax.axis_index("c")
    s = lax.axis_index("s")
    b1v, _ = _lane_consts(_NSUB1 * 8)

    z = jnp.zeros((_D,), jnp.float32)

    def zb(r, carry):
        val16_v[r] = z
        return carry

    lax.fori_loop(0, _C1, zb, 0)
    nz = (_NEZ // _NS) + jnp.where(s < _NEZ % _NS, 1, 0)

    def zbody(i, carry):
        pltpu.sync_copy(val16_v.at[pl.ds(0, _EZ)],
                        acc_sh.at[pl.ds((s + i * _NS) * _EZ, _EZ)])
        return carry

    lax.fori_loop(0, nz, zbody, 0)

    @pl.when(s < _NRB // _EZ)
    def _():
        pltpu.sync_copy(val16_v.at[pl.ds(0, _EZ)],
                        rel_sh.at[pl.ds(s * _EZ, _EZ)])

    plsc.subcore_barrier()

    nloc = (_NCH1 // _NS) + jnp.where(s < _NCH1 % _NS, 1, 0)

    def make_chunk_body(do_rel):
        def chunk_body(i, carry):
            cid = s + i * _NS
            base = cid * _C1
            tc0 = cid * _NSUB1
            idx_src = heads1 if do_rel else tails1
            cps = [
                pltpu.make_async_copy(val3.at[0, pl.ds(tc0 * 8, _NSUB1 * 8)],
                                      vv.at[pl.ds(0, _NSUB1 * 8)], sem),
                pltpu.make_async_copy(val3.at[1, pl.ds(tc0 * 8, _NSUB1 * 8)],
                                      vv.at[pl.ds(_NSUB1 * 8, _NSUB1 * 8)],
                                      sem),
                pltpu.make_async_copy(w1.at[pl.ds(base, _C1)], w_v, sem),
            ]
            cps += [
                pltpu.make_async_copy(
                    idx_src.at[pl.ds(base + j * 128, 128)],
                    sidx_v.at[j], sem)
                for j in range(_NSUB1)
            ]
            if do_rel:
                cps.append(pltpu.make_async_copy(
                    rels1.at[pl.ds(base, _C1)], rels_v, sem))
                cps.append(pltpu.make_async_copy(
                    ids1.at[pl.ds(base, _C1)], ids_v, sem))
            for cp in cps:
                cp.start()
            for cp in cps:
                cp.wait()

            def groupfn(g, rcarry):
                gb = g * 16
                w16 = w_v[pl.ds(gb, 16)]
                if do_rel:
                    ridx_v[g // 8, pl.ds((g % 8) * 16, 16)] = (
                        rels_v[pl.ds(gb, 16)] + ids_v[pl.ds(gb, 16)] * _NREL)
                row_v = b1v + (g // 8) * 8
                cs = _splat((g % 8) * 16)
                for r in range(16):
                    row = plsc.load_gather(vv, [row_v, cs + r])
                    val16_v[gb + r] = row * w16[r]
                return rcarry

            lax.fori_loop(0, _C1 // 16, groupfn, 0)

            scs = [pltpu.make_async_copy(val16_v.at[pl.ds(j * 128, 128)],
                                         acc_sh.at[sidx_v.at[j]], sem)
                   for j in range(_NSUB1)]
            if do_rel:
                scs += [pltpu.make_async_copy(
                    val16_v.at[pl.ds(j * 128, 128)],
                    rel_sh.at[ridx_v.at[j]], sem)
                    for j in range(_NSUB1)]
            for sc_ in scs:
                sc_.start(add=True)
            for sc_ in scs:
                sc_.wait()
            return carry

        return chunk_body

    @pl.when(c == 0)
    def _():
        lax.fori_loop(0, nloc, make_chunk_body(True), 0)

    @pl.when(c != 0)
    def _():
        lax.fori_loop(0, nloc, make_chunk_body(False), 0)

    plsc.subcore_barrier()

    def make_wb(dst):
        def wb(i, carry):
            b = (s + i * _NS) * _EZ
            pltpu.sync_copy(acc_sh.at[pl.ds(b, _EZ)], dst.at[pl.ds(b, _EZ)])
            return carry

        return wb

    @pl.when(c == 0)
    def _():
        lax.fori_loop(0, nz, make_wb(head_out), 0)

        @pl.when(s < _NRB // _EZ)
        def _():
            pltpu.sync_copy(rel_sh.at[pl.ds(s * _EZ, _EZ)],
                            rel_out.at[pl.ds(s * _EZ, _EZ)])

    @pl.when(c != 0)
    def _():
        lax.fori_loop(0, nz, make_wb(tail_out), 0)


def _p2_body(heads1, w1, head_in, tail_in, rel_in, out3,
             val_v, ov, w_v, gidx_v, sbuf, ov1, sem):
    c = lax.axis_index("c")
    s = lax.axis_index("s")
    w = s * _NC + c
    b2v, iv = _lane_consts(_NSUB2 * 8)
    b3v, _ = _lane_consts(8)

    ncp = (_HDRTC // _NW) + jnp.where(w < _HDRTC % _NW, 1, 0)

    def colfn(i, carry):
        t = w + i * _NW
        rbase = t * 128

        @pl.when(t < _HT_COL)
        def _():
            pltpu.sync_copy(head_in.at[pl.ds(rbase, 128)], sbuf)

        @pl.when(t == _HT_COL)
        def _():
            pltpu.sync_copy(head_in.at[pl.ds(_HT_COL * 128, _NE % 128)],
                            sbuf.at[pl.ds(0, _NE % 128)])
            pltpu.sync_copy(tail_in.at[pl.ds(0, 128 - _NE % 128)],
                            sbuf.at[pl.ds(_NE % 128, 128 - _NE % 128)])

        @pl.when(jnp.logical_and(t > _HT_COL, t < _TR_COL))
        def _():
            pltpu.sync_copy(tail_in.at[pl.ds(rbase - _NE, 128)], sbuf)

        @pl.when(t == _TR_COL)
        def _():
            pltpu.sync_copy(tail_in.at[pl.ds(_TR_COL * 128 - _NE, 64)],
                            sbuf.at[pl.ds(0, 64)])
            pltpu.sync_copy(rel_in.at[pl.ds(0, 64)], sbuf.at[pl.ds(64, 64)])

        @pl.when(t > _TR_COL)
        def _():
            pltpu.sync_copy(rel_in.at[pl.ds(rbase - 2 * _NE, 128)], sbuf)

        def tb(g, rcarry):
            gb = g * 16
            cs = _splat(gb)
            for r in range(16):
                plsc.store_scatter(ov1, [b3v, cs + r], sbuf[gb + r])
            return rcarry

        lax.fori_loop(0, 8, tb, 0)
        pltpu.sync_copy(ov1.at[pl.ds(0, 8)], out3.at[0, pl.ds(t * 8, 8)])
        pltpu.sync_copy(ov1.at[pl.ds(8, 8)], out3.at[1, pl.ds(t * 8, 8)])
        return carry

    lax.fori_loop(0, ncp, colfn, 0)

    nloc = (_NCH2 // _NW) + jnp.where(w < _NCH2 % _NW, 1, 0)

    def chunk_body(i, carry):
        cid = w + i * _NW
        base = cid * _C2
        cps = [pltpu.make_async_copy(w1.at[pl.ds(base, _C2)], w_v, sem)]
        cps += [
            pltpu.make_async_copy(heads1.at[pl.ds(base + j * 128, 128)],
                                  gidx_v.at[j], sem)
            for j in range(_NSUB2)
        ]
        for cp in cps:
            cp.start()
        for cp in cps:
            cp.wait()

        gs = [pltpu.make_async_copy(head_in.at[gidx_v.at[j]],
                                    val_v.at[pl.ds(j * 128, 128)], sem)
              for j in range(_NSUB2)]
        for g in gs:
            g.start()
        for g in gs:
            g.wait()

        def groupfn(g, rcarry):
            gb = g * 16
            w16 = w_v[pl.ds(gb, 16)]
            row_v = b2v + (g // 8) * 8
            cs = _splat((g % 8) * 16)
            for r in range(16):
                row = val_v[gb + r] * w16[r]
                plsc.store_scatter(ov, [row_v, cs + r], row)
            return rcarry

        lax.fori_loop(0, _C2 // 16, groupfn, 0)
        mo = (_HDRTC + cid * _NSUB2) * 8
        nm = _NSUB2 * 8
        pltpu.sync_copy(ov.at[pl.ds(0, nm)], out3.at[0, pl.ds(mo, nm)])
        pltpu.sync_copy(ov.at[pl.ds(nm, nm)], out3.at[1, pl.ds(mo, nm)])
        return carry

    lax.fori_loop(0, nloc, chunk_body, 0)


def kernel(batch_heads, batch_rels, batch_tails, batch_ids, fact_ids,
           weight_list, fact_val):
    del fact_ids
    val3 = fact_val.reshape(_VTC, 128, 2, 8).transpose(2, 0, 3, 1).reshape(
        2, _NF // 16, 128)
    mesh = plsc.VectorSubcoreMesh(core_axis_name="c", subcore_axis_name="s")

    f32 = jnp.float32
    i32 = jnp.int32
    cparams = pltpu.CompilerParams(use_tc_tiling_on_sc=False,
                                   needs_layout_passes=False)
    p1 = pl.kernel(
        _p1_body,
        out_type=(
            jax.ShapeDtypeStruct((_NE, _D), f32),
            jax.ShapeDtypeStruct((_NE, _D), f32),
            jax.ShapeDtypeStruct((_NRB, _D), f32),
        ),
        mesh=mesh,
        scratch_types=[
            pltpu.VMEM_SHARED((_NE, _D), f32),
            pltpu.VMEM_SHARED((_NRB, _D), f32),
            pltpu.VMEM((2 * _NSUB1 * 8, 128), f32),
            pltpu.VMEM((_C1, _D), f32),
            pltpu.VMEM((_C1,), f32),
            pltpu.VMEM((_NSUB1, 128), i32),
            pltpu.VMEM((_C1,), i32),
            pltpu.VMEM((_C1,), i32),
            pltpu.VMEM((_NSUB1, 128), i32),
            pltpu.SemaphoreType.DMA,
        ],
        compiler_params=cparams,
    )
    head_agg, tail_agg, rel_agg = p1(batch_heads, batch_tails, batch_rels,
                                     batch_ids, weight_list, val3)

    p2 = pl.kernel(
        _p2_body,
        out_type=jax.ShapeDtypeStruct((2, _OTC * 8, 128), f32),
        mesh=mesh,
        scratch_types=[
            pltpu.VMEM((_C2, _D), f32),
            pltpu.VMEM((2 * _NSUB2 * 8, 128), f32),
            pltpu.VMEM((_C2,), f32),
            pltpu.VMEM((_NSUB2, 128), i32),
            pltpu.VMEM((128, _D), f32),
            pltpu.VMEM((16, 128), f32),
            pltpu.SemaphoreType.DMA,
        ],
        compiler_params=cparams,
    )
    out3 = p2(batch_heads, weight_list, head_agg, tail_agg, rel_agg)
    return out3.reshape(2, _OTC, 8, 128).transpose(1, 3, 0, 2).reshape(
        _OUT_ROWS, _D)


# speedup vs baseline: 2.0115x; 1.5732x over previous
import jax
import jax.numpy as jnp
from jax import lax
from jax.experimental import pallas as pl
from jax.experimental.pallas import tpu as pltpu, tpu_sc as plsc

_NE = 100_000
_NRB = 1_600
_NF = 3_200_000
_D = 16
_NREL = 200
_NC = 2
_NS = 16
_NW = _NC * _NS
_OUT_ROWS = 2 * _NE + _NRB + _NF
_OTC = _OUT_ROWS // 128
_VTC = _NF // 128
_HDRTC = (2 * _NE + _NRB) // 128
_HT_COL = _NE // 128
_TR_COL = 2 * _NE // 128

_C1 = 512
_NSUB1 = _C1 // 128
_NCH1 = _NF // _C1
_C2 = 1024
_NSUB2 = _C2 // 128
_NCH2 = _NF // _C2
_EZ = 400
_NEZ = _NE // _EZ


def _lane_consts(stride):
    iota = lax.iota(jnp.int32, 16)
    return (iota // 8) * stride + iota % 8, iota


def _splat(x):
    return jnp.broadcast_to(x, (16,))


def _p1_body(heads1, tails1, rels1, ids1, w1, val3,
             head_out, tail_out, rel_out,
             acc_sh, rel_sh, vv, val16_v, w_v, sidx_v, rels_v, ids_v,
             ridx_v, sem):
    c = lax.axis_index("c")
    s = lax.axis_index("s")
    b1v, _ = _lane_consts(_NSUB1 * 8)

    z = jnp.zeros((_D,), jnp.float32)

    def zb(r, carry):
        val16_v[r] = z
        return carry

    lax.fori_loop(0, _C1, zb, 0)
    nz = (_NEZ // _NS) + jnp.where(s < _NEZ % _NS, 1, 0)

    def zbody(i, carry):
        pltpu.sync_copy(val16_v.at[pl.ds(0, _EZ)],
                        acc_sh.at[pl.ds((s + i * _NS) * _EZ, _EZ)])
        return carry

    lax.fori_loop(0, nz, zbody, 0)

    @pl.when(s < _NRB // _EZ)
    def _():
        pltpu.sync_copy(val16_v.at[pl.ds(0, _EZ)],
                        rel_sh.at[pl.ds(s * _EZ, _EZ)])

    plsc.subcore_barrier()

    nloc = (_NCH1 // _NS) + jnp.where(s < _NCH1 % _NS, 1, 0)

    def make_chunk_body(do_rel):
        def chunk_body(i, carry):
            cid = s + i * _NS
            base = cid * _C1
            tc0 = cid * _NSUB1
            idx_src = heads1 if do_rel else tails1
            cps = [
                pltpu.make_async_copy(val3.at[0, pl.ds(tc0 * 8, _NSUB1 * 8)],
                                      vv.at[pl.ds(0, _NSUB1 * 8)], sem),
                pltpu.make_async_copy(val3.at[1, pl.ds(tc0 * 8, _NSUB1 * 8)],
                                      vv.at[pl.ds(_NSUB1 * 8, _NSUB1 * 8)],
                                      sem),
                pltpu.make_async_copy(w1.at[pl.ds(base, _C1)], w_v, sem),
            ]
            cps += [
                pltpu.make_async_copy(
                    idx_src.at[pl.ds(base + j * 128, 128)],
                    sidx_v.at[j], sem)
                for j in range(_NSUB1)
            ]
            if do_rel:
                cps.append(pltpu.make_async_copy(
                    rels1.at[pl.ds(base, _C1)], rels_v, sem))
                cps.append(pltpu.make_async_copy(
                    ids1.at[pl.ds(base, _C1)], ids_v, sem))
            for cp in cps:
                cp.start()
            for cp in cps:
                cp.wait()

            @plsc.parallel_loop(0, _C1 // 16, 1)
            def _(g):
                gb = g * 16
                w16 = w_v[pl.ds(gb, 16)]
                if do_rel:
                    ridx_v[g // 8, pl.ds((g % 8) * 16, 16)] = (
                        rels_v[pl.ds(gb, 16)] + ids_v[pl.ds(gb, 16)] * _NREL)
                row_v = b1v + (g // 8) * 8
                cs = _splat((g % 8) * 16)
                for r in range(16):
                    row = plsc.load_gather(vv, [row_v, cs + r])
                    val16_v[gb + r] = row * w16[r]

            scs = [pltpu.make_async_copy(val16_v.at[pl.ds(j * 128, 128)],
                                         acc_sh.at[sidx_v.at[j]], sem)
                   for j in range(_NSUB1)]
            if do_rel:
                scs += [pltpu.make_async_copy(
                    val16_v.at[pl.ds(j * 128, 128)],
                    rel_sh.at[ridx_v.at[j]], sem)
                    for j in range(_NSUB1)]
            for sc_ in scs:
                sc_.start(add=True)
            for sc_ in scs:
                sc_.wait()
            return carry

        return chunk_body

    @pl.when(c == 0)
    def _():
        lax.fori_loop(0, nloc, make_chunk_body(True), 0)

    @pl.when(c != 0)
    def _():
        lax.fori_loop(0, nloc, make_chunk_body(False), 0)

    plsc.subcore_barrier()

    def make_wb(dst):
        def wb(i, carry):
            b = (s + i * _NS) * _EZ
            pltpu.sync_copy(acc_sh.at[pl.ds(b, _EZ)], dst.at[pl.ds(b, _EZ)])
            return carry

        return wb

    @pl.when(c == 0)
    def _():
        lax.fori_loop(0, nz, make_wb(head_out), 0)

        @pl.when(s < _NRB // _EZ)
        def _():
            pltpu.sync_copy(rel_sh.at[pl.ds(s * _EZ, _EZ)],
                            rel_out.at[pl.ds(s * _EZ, _EZ)])

    @pl.when(c != 0)
    def _():
        lax.fori_loop(0, nz, make_wb(tail_out), 0)


def _p2_body(heads1, w1, head_in, tail_in, rel_in, out3,
             val_v, ov, w_v, gidx_v, sbuf, ov1, sem):
    c = lax.axis_index("c")
    s = lax.axis_index("s")
    w = s * _NC + c
    b2v, iv = _lane_consts(_NSUB2 * 8)
    b3v, _ = _lane_consts(8)

    ncp = (_HDRTC // _NW) + jnp.where(w < _HDRTC % _NW, 1, 0)

    def colfn(i, carry):
        t = w + i * _NW
        rbase = t * 128

        @pl.when(t < _HT_COL)
        def _():
            pltpu.sync_copy(head_in.at[pl.ds(rbase, 128)], sbuf)

        @pl.when(t == _HT_COL)
        def _():
            pltpu.sync_copy(head_in.at[pl.ds(_HT_COL * 128, _NE % 128)],
                            sbuf.at[pl.ds(0, _NE % 128)])
            pltpu.sync_copy(tail_in.at[pl.ds(0, 128 - _NE % 128)],
                            sbuf.at[pl.ds(_NE % 128, 128 - _NE % 128)])

        @pl.when(jnp.logical_and(t > _HT_COL, t < _TR_COL))
        def _():
            pltpu.sync_copy(tail_in.at[pl.ds(rbase - _NE, 128)], sbuf)

        @pl.when(t == _TR_COL)
        def _():
            pltpu.sync_copy(tail_in.at[pl.ds(_TR_COL * 128 - _NE, 64)],
                            sbuf.at[pl.ds(0, 64)])
            pltpu.sync_copy(rel_in.at[pl.ds(0, 64)], sbuf.at[pl.ds(64, 64)])

        @pl.when(t > _TR_COL)
        def _():
            pltpu.sync_copy(rel_in.at[pl.ds(rbase - 2 * _NE, 128)], sbuf)

        @plsc.parallel_loop(0, 8, 1)
        def _(g):
            gb = g * 16
            cs = _splat(gb)
            for r in range(16):
                plsc.store_scatter(ov1, [b3v, cs + r], sbuf[gb + r])
        pltpu.sync_copy(ov1.at[pl.ds(0, 8)], out3.at[0, pl.ds(t * 8, 8)])
        pltpu.sync_copy(ov1.at[pl.ds(8, 8)], out3.at[1, pl.ds(t * 8, 8)])
        return carry

    lax.fori_loop(0, ncp, colfn, 0)

    nloc = (_NCH2 // _NW) + jnp.where(w < _NCH2 % _NW, 1, 0)

    def chunk_body(i, carry):
        cid = w + i * _NW
        base = cid * _C2
        cps = [pltpu.make_async_copy(w1.at[pl.ds(base, _C2)], w_v, sem)]
        cps += [
            pltpu.make_async_copy(heads1.at[pl.ds(base + j * 128, 128)],
                                  gidx_v.at[j], sem)
            for j in range(_NSUB2)
        ]
        for cp in cps:
            cp.start()
        for cp in cps:
            cp.wait()

        gs = [pltpu.make_async_copy(head_in.at[gidx_v.at[j]],
                                    val_v.at[pl.ds(j * 128, 128)], sem)
              for j in range(_NSUB2)]
        for g in gs:
            g.start()
        for g in gs:
            g.wait()

        @plsc.parallel_loop(0, _C2 // 16, 1)
        def _(g):
            gb = g * 16
            w16 = w_v[pl.ds(gb, 16)]
            row_v = b2v + (g // 8) * 8
            cs = _splat((g % 8) * 16)
            for r in range(16):
                row = val_v[gb + r] * w16[r]
                plsc.store_scatter(ov, [row_v, cs + r], row)
        mo = (_HDRTC + cid * _NSUB2) * 8
        nm = _NSUB2 * 8
        pltpu.sync_copy(ov.at[pl.ds(0, nm)], out3.at[0, pl.ds(mo, nm)])
        pltpu.sync_copy(ov.at[pl.ds(nm, nm)], out3.at[1, pl.ds(mo, nm)])
        return carry

    lax.fori_loop(0, nloc, chunk_body, 0)


def kernel(batch_heads, batch_rels, batch_tails, batch_ids, fact_ids,
           weight_list, fact_val):
    del fact_ids
    val3 = fact_val.reshape(_VTC, 128, 2, 8).transpose(2, 0, 3, 1).reshape(
        2, _NF // 16, 128)
    mesh = plsc.VectorSubcoreMesh(core_axis_name="c", subcore_axis_name="s")

    f32 = jnp.float32
    i32 = jnp.int32
    cparams = pltpu.CompilerParams(use_tc_tiling_on_sc=False,
                                   needs_layout_passes=False)
    p1 = pl.kernel(
        _p1_body,
        out_type=(
            jax.ShapeDtypeStruct((_NE, _D), f32),
            jax.ShapeDtypeStruct((_NE, _D), f32),
            jax.ShapeDtypeStruct((_NRB, _D), f32),
        ),
        mesh=mesh,
        scratch_types=[
            pltpu.VMEM_SHARED((_NE, _D), f32),
            pltpu.VMEM_SHARED((_NRB, _D), f32),
            pltpu.VMEM((2 * _NSUB1 * 8, 128), f32),
            pltpu.VMEM((_C1, _D), f32),
            pltpu.VMEM((_C1,), f32),
            pltpu.VMEM((_NSUB1, 128), i32),
            pltpu.VMEM((_C1,), i32),
            pltpu.VMEM((_C1,), i32),
            pltpu.VMEM((_NSUB1, 128), i32),
            pltpu.SemaphoreType.DMA,
        ],
        compiler_params=cparams,
    )
    head_agg, tail_agg, rel_agg = p1(batch_heads, batch_tails, batch_rels,
                                     batch_ids, weight_list, val3)

    p2 = pl.kernel(
        _p2_body,
        out_type=jax.ShapeDtypeStruct((2, _OTC * 8, 128), f32),
        mesh=mesh,
        scratch_types=[
            pltpu.VMEM((_C2, _D), f32),
            pltpu.VMEM((2 * _NSUB2 * 8, 128), f32),
            pltpu.VMEM((_C2,), f32),
            pltpu.VMEM((_NSUB2, 128), i32),
            pltpu.VMEM((128, _D), f32),
            pltpu.VMEM((16, 128), f32),
            pltpu.SemaphoreType.DMA,
        ],
        compiler_params=cparams,
    )
    out3 = p2(batch_heads, weight_list, head_agg, tail_agg, rel_agg)
    return out3.reshape(2, _OTC, 8, 128).transpose(1, 3, 0, 2).reshape(
        _OUT_ROWS, _D)


# speedup vs baseline: 4.2988x; 2.1371x over previous
import jax
import jax.numpy as jnp
from jax import lax
from jax.experimental import pallas as pl
from jax.experimental.pallas import tpu as pltpu, tpu_sc as plsc

_NE = 100_000
_NRB = 1_600
_NF = 3_200_000
_D = 16
_NREL = 200
_NC = 2
_NS = 16
_NW = _NC * _NS
_OUT_ROWS = 2 * _NE + _NRB + _NF
_OTC = _OUT_ROWS // 128
_VTC = _NF // 128
_HDRTC = (2 * _NE + _NRB) // 128
_HT_COL = _NE // 128
_TR_COL = 2 * _NE // 128

_C1 = 512
_NSUB1 = _C1 // 128
_NCH1 = _NF // _C1
_C2 = 1024
_NSUB2 = _C2 // 128
_NCH2 = _NF // _C2
_EZ = 400
_NEZ = _NE // _EZ


def _lane_consts(stride):
    iota = lax.iota(jnp.int32, 16)
    return (iota // 8) * stride + iota % 8, iota


def _splat(x):
    return jnp.broadcast_to(x, (16,))


def _p1_body(heads1, tails1, rels1, ids1, w1, val3,
             head_out, tail_out, rel_out,
             acc_sh, rel_sh, vv, val16_v, w_v, sidx_v, rels_v, ids_v,
             ridx_v, sem):
    c = lax.axis_index("c")
    s = lax.axis_index("s")
    b1v, _ = _lane_consts(_NSUB1 * 8)

    z = jnp.zeros((_D,), jnp.float32)

    def zb(r, carry):
        val16_v[r] = z
        return carry

    lax.fori_loop(0, _C1, zb, 0)
    nz = (_NEZ // _NS) + jnp.where(s < _NEZ % _NS, 1, 0)

    def zbody(i, carry):
        pltpu.sync_copy(val16_v.at[pl.ds(0, _EZ)],
                        acc_sh.at[pl.ds((s + i * _NS) * _EZ, _EZ)])
        return carry

    lax.fori_loop(0, nz, zbody, 0)

    @pl.when(s < _NRB // _EZ)
    def _():
        pltpu.sync_copy(val16_v.at[pl.ds(0, _EZ)],
                        rel_sh.at[pl.ds(s * _EZ, _EZ)])

    plsc.subcore_barrier()

    nloc = (_NCH1 // _NS) + jnp.where(s < _NCH1 % _NS, 1, 0)

    def make_chunk_body(do_rel):
        def chunk_body(i, carry):
            cid = s + i * _NS
            base = cid * _C1
            tc0 = cid * _NSUB1
            idx_src = heads1 if do_rel else tails1
            cps = [
                pltpu.make_async_copy(val3.at[0, pl.ds(tc0 * 8, _NSUB1 * 8)],
                                      vv.at[pl.ds(0, _NSUB1 * 8),
                                            pl.ds(0, 128)], sem),
                pltpu.make_async_copy(val3.at[1, pl.ds(tc0 * 8, _NSUB1 * 8)],
                                      vv.at[pl.ds(_NSUB1 * 8, _NSUB1 * 8),
                                            pl.ds(0, 128)], sem),
                pltpu.make_async_copy(w1.at[pl.ds(base, _C1)], w_v, sem),
            ]
            cps += [
                pltpu.make_async_copy(
                    idx_src.at[pl.ds(base + j * 128, 128)],
                    sidx_v.at[j], sem)
                for j in range(_NSUB1)
            ]
            if do_rel:
                cps.append(pltpu.make_async_copy(
                    rels1.at[pl.ds(base, _C1)], rels_v, sem))
                cps.append(pltpu.make_async_copy(
                    ids1.at[pl.ds(base, _C1)], ids_v, sem))
            for cp in cps:
                cp.start()
            for cp in cps:
                cp.wait()

            @plsc.parallel_loop(0, _C1 // 16, 1)
            def _(g):
                gb = g * 16
                w16 = w_v[pl.ds(gb, 16)]
                if do_rel:
                    ridx_v[g // 8, pl.ds((g % 8) * 16, 16)] = (
                        rels_v[pl.ds(gb, 16)] + ids_v[pl.ds(gb, 16)] * _NREL)
                row_v = b1v + (g // 8) * 8
                cs = _splat((g % 8) * 16)
                for r in range(16):
                    row = plsc.load_gather(vv, [row_v, cs + r])
                    val16_v[gb + r] = row * w16[r]

            scs = [pltpu.make_async_copy(val16_v.at[pl.ds(j * 128, 128)],
                                         acc_sh.at[sidx_v.at[j]], sem)
                   for j in range(_NSUB1)]
            if do_rel:
                scs += [pltpu.make_async_copy(
                    val16_v.at[pl.ds(j * 128, 128)],
                    rel_sh.at[ridx_v.at[j]], sem)
                    for j in range(_NSUB1)]
            for sc_ in scs:
                sc_.start(add=True)
            for sc_ in scs:
                sc_.wait()
            return carry

        return chunk_body

    @pl.when(c == 0)
    def _():
        lax.fori_loop(0, nloc, make_chunk_body(True), 0)

    @pl.when(c != 0)
    def _():
        lax.fori_loop(0, nloc, make_chunk_body(False), 0)

    plsc.subcore_barrier()

    def make_wb(dst):
        def wb(i, carry):
            b = (s + i * _NS) * _EZ
            pltpu.sync_copy(acc_sh.at[pl.ds(b, _EZ)], dst.at[pl.ds(b, _EZ)])
            return carry

        return wb

    @pl.when(c == 0)
    def _():
        lax.fori_loop(0, nz, make_wb(head_out), 0)

        @pl.when(s < _NRB // _EZ)
        def _():
            pltpu.sync_copy(rel_sh.at[pl.ds(s * _EZ, _EZ)],
                            rel_out.at[pl.ds(s * _EZ, _EZ)])

    @pl.when(c != 0)
    def _():
        lax.fori_loop(0, nz, make_wb(tail_out), 0)


def _p2_body(heads1, w1, head_in, tail_in, rel_in, out3,
             val_v, ov, w_v, gidx_v, sbuf, ov1, sem):
    c = lax.axis_index("c")
    s = lax.axis_index("s")
    w = s * _NC + c
    b2v, iv = _lane_consts(_NSUB2 * 8)
    b3v, _ = _lane_consts(8)

    ncp = (_HDRTC // _NW) + jnp.where(w < _HDRTC % _NW, 1, 0)

    def colfn(i, carry):
        t = w + i * _NW
        rbase = t * 128

        @pl.when(t < _HT_COL)
        def _():
            pltpu.sync_copy(head_in.at[pl.ds(rbase, 128)], sbuf)

        @pl.when(t == _HT_COL)
        def _():
            pltpu.sync_copy(head_in.at[pl.ds(_HT_COL * 128, _NE % 128)],
                            sbuf.at[pl.ds(0, _NE % 128)])
            pltpu.sync_copy(tail_in.at[pl.ds(0, 128 - _NE % 128)],
                            sbuf.at[pl.ds(_NE % 128, 128 - _NE % 128)])

        @pl.when(jnp.logical_and(t > _HT_COL, t < _TR_COL))
        def _():
            pltpu.sync_copy(tail_in.at[pl.ds(rbase - _NE, 128)], sbuf)

        @pl.when(t == _TR_COL)
        def _():
            pltpu.sync_copy(tail_in.at[pl.ds(_TR_COL * 128 - _NE, 64)],
                            sbuf.at[pl.ds(0, 64)])
            pltpu.sync_copy(rel_in.at[pl.ds(0, 64)], sbuf.at[pl.ds(64, 64)])

        @pl.when(t > _TR_COL)
        def _():
            pltpu.sync_copy(rel_in.at[pl.ds(rbase - 2 * _NE, 128)], sbuf)

        @plsc.parallel_loop(0, 8, 1)
        def _(g):
            gb = g * 16
            cs = _splat(gb)
            for r in range(16):
                plsc.store_scatter(ov1, [b3v, cs + r], sbuf[gb + r])

        pltpu.sync_copy(ov1.at[pl.ds(0, 8), pl.ds(0, 128)],
                        out3.at[0, pl.ds(t * 8, 8)])
        pltpu.sync_copy(ov1.at[pl.ds(8, 8), pl.ds(0, 128)],
                        out3.at[1, pl.ds(t * 8, 8)])
        return carry

    lax.fori_loop(0, ncp, colfn, 0)

    nloc = (_NCH2 // _NW) + jnp.where(w < _NCH2 % _NW, 1, 0)

    def chunk_body(i, carry):
        cid = w + i * _NW
        base = cid * _C2
        cps = [pltpu.make_async_copy(w1.at[pl.ds(base, _C2)], w_v, sem)]
        cps += [
            pltpu.make_async_copy(heads1.at[pl.ds(base + j * 128, 128)],
                                  gidx_v.at[j], sem)
            for j in range(_NSUB2)
        ]
        for cp in cps:
            cp.start()
        for cp in cps:
            cp.wait()

        gs = [pltpu.make_async_copy(head_in.at[gidx_v.at[j]],
                                    val_v.at[pl.ds(j * 128, 128)], sem)
              for j in range(_NSUB2)]
        for g in gs:
            g.start()
        for g in gs:
            g.wait()

        @plsc.parallel_loop(0, _C2 // 16, 1)
        def _(g):
            gb = g * 16
            w16 = w_v[pl.ds(gb, 16)]
            row_v = b2v + (g // 8) * 8
            cs = _splat((g % 8) * 16)
            for r in range(16):
                row = val_v[gb + r] * w16[r]
                plsc.store_scatter(ov, [row_v, cs + r], row)
        mo = (_HDRTC + cid * _NSUB2) * 8
        nm = _NSUB2 * 8
        pltpu.sync_copy(ov.at[pl.ds(0, nm), pl.ds(0, 128)],
                        out3.at[0, pl.ds(mo, nm)])
        pltpu.sync_copy(ov.at[pl.ds(nm, nm), pl.ds(0, 128)],
                        out3.at[1, pl.ds(mo, nm)])
        return carry

    lax.fori_loop(0, nloc, chunk_body, 0)


def kernel(batch_heads, batch_rels, batch_tails, batch_ids, fact_ids,
           weight_list, fact_val):
    del fact_ids
    val3 = fact_val.reshape(_VTC, 128, 2, 8).transpose(2, 0, 3, 1).reshape(
        2, _NF // 16, 128)
    mesh = plsc.VectorSubcoreMesh(core_axis_name="c", subcore_axis_name="s")

    f32 = jnp.float32
    i32 = jnp.int32
    cparams = pltpu.CompilerParams(use_tc_tiling_on_sc=False,
                                   needs_layout_passes=False)
    p1 = pl.kernel(
        _p1_body,
        out_type=(
            jax.ShapeDtypeStruct((_NE, _D), f32),
            jax.ShapeDtypeStruct((_NE, _D), f32),
            jax.ShapeDtypeStruct((_NRB, _D), f32),
        ),
        mesh=mesh,
        scratch_types=[
            pltpu.VMEM_SHARED((_NE, _D), f32),
            pltpu.VMEM_SHARED((_NRB, _D), f32),
            pltpu.VMEM((2 * _NSUB1 * 8, 129), f32),
            pltpu.VMEM((_C1, _D), f32),
            pltpu.VMEM((_C1,), f32),
            pltpu.VMEM((_NSUB1, 128), i32),
            pltpu.VMEM((_C1,), i32),
            pltpu.VMEM((_C1,), i32),
            pltpu.VMEM((_NSUB1, 128), i32),
            pltpu.SemaphoreType.DMA,
        ],
        compiler_params=cparams,
    )
    head_agg, tail_agg, rel_agg = p1(batch_heads, batch_tails, batch_rels,
                                     batch_ids, weight_list, val3)

    p2 = pl.kernel(
        _p2_body,
        out_type=jax.ShapeDtypeStruct((2, _OTC * 8, 128), f32),
        mesh=mesh,
        scratch_types=[
            pltpu.VMEM((_C2, _D), f32),
            pltpu.VMEM((2 * _NSUB2 * 8, 129), f32),
            pltpu.VMEM((_C2,), f32),
            pltpu.VMEM((_NSUB2, 128), i32),
            pltpu.VMEM((128, _D), f32),
            pltpu.VMEM((16, 129), f32),
            pltpu.SemaphoreType.DMA,
        ],
        compiler_params=cparams,
    )
    out3 = p2(batch_heads, weight_list, head_agg, tail_agg, rel_agg)
    return out3.reshape(2, _OTC, 8, 128).transpose(1, 3, 0, 2).reshape(
        _OUT_ROWS, _D)


# speedup vs baseline: 4.3860x; 1.0203x over previous
import jax
import jax.numpy as jnp
from jax import lax
from jax.experimental import pallas as pl
from jax.experimental.pallas import tpu as pltpu, tpu_sc as plsc

_NE = 100_000
_NRB = 1_600
_NF = 3_200_000
_D = 16
_NREL = 200
_NC = 2
_NS = 16
_NW = _NC * _NS
_OUT_ROWS = 2 * _NE + _NRB + _NF
_OTC = _OUT_ROWS // 128
_VTC = _NF // 128
_HDRTC = (2 * _NE + _NRB) // 128
_HT_COL = _NE // 128
_TR_COL = 2 * _NE // 128

_C1 = 512
_NSUB1 = _C1 // 128
_NCH1 = _NF // _C1
_C2 = 1024
_NSUB2 = _C2 // 128
_NCH2 = _NF // _C2
_EZ = 400
_NEZ = _NE // _EZ


def _lane_consts(stride):
    iota = lax.iota(jnp.int32, 16)
    return (iota // 8) * stride + iota % 8, iota


def _splat(x):
    return jnp.broadcast_to(x, (16,))


def _p1_body(heads1, tails1, rels1, ids1, w1, val3,
             head_out, tail_out, rel_out,
             acc_sh, rel_sh, vv0, val16_v, w_v0, sidx_v0, rels_v, ids_v,
             ridx_v, vv1, w_v1, sidx_v1, sem, sem2):
    c = lax.axis_index("c")
    s = lax.axis_index("s")
    b1v, _ = _lane_consts(_NSUB1 * 8)
    slots = ((vv0, w_v0, sidx_v0), (vv1, w_v1, sidx_v1))

    z = jnp.zeros((_D,), jnp.float32)

    def zb(r, carry):
        val16_v[r] = z
        return carry

    lax.fori_loop(0, _C1, zb, 0)
    nz = (_NEZ // _NS) + jnp.where(s < _NEZ % _NS, 1, 0)

    def zbody(i, carry):
        pltpu.sync_copy(val16_v.at[pl.ds(0, _EZ)],
                        acc_sh.at[pl.ds((s + i * _NS) * _EZ, _EZ)])
        return carry

    lax.fori_loop(0, nz, zbody, 0)

    @pl.when(s < _NRB // _EZ)
    def _():
        pltpu.sync_copy(val16_v.at[pl.ds(0, _EZ)],
                        rel_sh.at[pl.ds(s * _EZ, _EZ)])

    plsc.subcore_barrier()

    nloc = (_NCH1 // _NS) + jnp.where(s < _NCH1 % _NS, 1, 0)

    def run_core(do_rel):
        idx_src = heads1 if do_rel else tails1

        def stage_cps(slot, i):
            vv, w_v, sidx_v = slots[slot]
            cid = s + i * _NS
            base = cid * _C1
            tc0 = cid * _NSUB1
            cps = [
                pltpu.make_async_copy(val3.at[0, pl.ds(tc0 * 8, _NSUB1 * 8)],
                                      vv.at[pl.ds(0, _NSUB1 * 8),
                                            pl.ds(0, 128)], sem),
                pltpu.make_async_copy(val3.at[1, pl.ds(tc0 * 8, _NSUB1 * 8)],
                                      vv.at[pl.ds(_NSUB1 * 8, _NSUB1 * 8),
                                            pl.ds(0, 128)], sem),
                pltpu.make_async_copy(w1.at[pl.ds(base, _C1)], w_v, sem),
            ]
            cps += [
                pltpu.make_async_copy(
                    idx_src.at[pl.ds(base + j * 128, 128)],
                    sidx_v.at[j], sem)
                for j in range(_NSUB1)
            ]
            return cps

        def start_stage(slot, i):
            for cp in stage_cps(slot, i):
                cp.start()

        def compute_scatter(slot, i):
            vv, w_v, sidx_v = slots[slot]
            if do_rel:
                base = (s + i * _NS) * _C1
                rcs = [pltpu.make_async_copy(rels1.at[pl.ds(base, _C1)],
                                             rels_v, sem2),
                       pltpu.make_async_copy(ids1.at[pl.ds(base, _C1)],
                                             ids_v, sem2)]
                for cp in rcs:
                    cp.start()
                for cp in rcs:
                    cp.wait()
            for cp in stage_cps(slot, i):
                cp.wait()

            @plsc.parallel_loop(0, _C1 // 16, 1)
            def _(g):
                gb = g * 16
                w16 = w_v[pl.ds(gb, 16)]
                if do_rel:
                    ridx_v[g // 8, pl.ds((g % 8) * 16, 16)] = (
                        rels_v[pl.ds(gb, 16)] + ids_v[pl.ds(gb, 16)] * _NREL)
                row_v = b1v + (g // 8) * 8
                cs = _splat((g % 8) * 16)
                for r in range(16):
                    row = plsc.load_gather(vv, [row_v, cs + r])
                    val16_v[gb + r] = row * w16[r]

            scs = [pltpu.make_async_copy(val16_v.at[pl.ds(j * 128, 128)],
                                         acc_sh.at[sidx_v.at[j]], sem2)
                   for j in range(_NSUB1)]
            if do_rel:
                scs += [pltpu.make_async_copy(
                    val16_v.at[pl.ds(j * 128, 128)],
                    rel_sh.at[ridx_v.at[j]], sem2)
                    for j in range(_NSUB1)]
            for sc_ in scs:
                sc_.start(add=True)
            for sc_ in scs:
                sc_.wait()

        start_stage(0, 0)

        def pair_body(h, carry):
            a = h * 2
            b = a + 1

            @pl.when(b < nloc)
            def _():
                start_stage(1, b)

            compute_scatter(0, a)

            @pl.when(a + 2 < nloc)
            def _():
                start_stage(0, a + 2)

            @pl.when(b < nloc)
            def _():
                compute_scatter(1, b)

            return carry

        lax.fori_loop(0, (nloc + 1) // 2, pair_body, 0)

    @pl.when(c == 0)
    def _():
        run_core(True)

    @pl.when(c != 0)
    def _():
        run_core(False)

    plsc.subcore_barrier()

    def make_wb(dst):
        def wb(i, carry):
            b = (s + i * _NS) * _EZ
            pltpu.sync_copy(acc_sh.at[pl.ds(b, _EZ)], dst.at[pl.ds(b, _EZ)])
            return carry

        return wb

    @pl.when(c == 0)
    def _():
        lax.fori_loop(0, nz, make_wb(head_out), 0)

        @pl.when(s < _NRB // _EZ)
        def _():
            pltpu.sync_copy(rel_sh.at[pl.ds(s * _EZ, _EZ)],
                            rel_out.at[pl.ds(s * _EZ, _EZ)])

    @pl.when(c != 0)
    def _():
        lax.fori_loop(0, nz, make_wb(tail_out), 0)


def _p2_body(heads1, w1, head_in, tail_in, rel_in, out3,
             val_v, ov, w_v, gidx_v, sbuf, ov1, sem):
    c = lax.axis_index("c")
    s = lax.axis_index("s")
    w = s * _NC + c
    b2v, iv = _lane_consts(_NSUB2 * 8)
    b3v, _ = _lane_consts(8)

    ncp = (_HDRTC // _NW) + jnp.where(w < _HDRTC % _NW, 1, 0)

    def colfn(i, carry):
        t = w + i * _NW
        rbase = t * 128

        @pl.when(t < _HT_COL)
        def _():
            pltpu.sync_copy(head_in.at[pl.ds(rbase, 128)], sbuf)

        @pl.when(t == _HT_COL)
        def _():
            pltpu.sync_copy(head_in.at[pl.ds(_HT_COL * 128, _NE % 128)],
                            sbuf.at[pl.ds(0, _NE % 128)])
            pltpu.sync_copy(tail_in.at[pl.ds(0, 128 - _NE % 128)],
                            sbuf.at[pl.ds(_NE % 128, 128 - _NE % 128)])

        @pl.when(jnp.logical_and(t > _HT_COL, t < _TR_COL))
        def _():
            pltpu.sync_copy(tail_in.at[pl.ds(rbase - _NE, 128)], sbuf)

        @pl.when(t == _TR_COL)
        def _():
            pltpu.sync_copy(tail_in.at[pl.ds(_TR_COL * 128 - _NE, 64)],
                            sbuf.at[pl.ds(0, 64)])
            pltpu.sync_copy(rel_in.at[pl.ds(0, 64)], sbuf.at[pl.ds(64, 64)])

        @pl.when(t > _TR_COL)
        def _():
            pltpu.sync_copy(rel_in.at[pl.ds(rbase - 2 * _NE, 128)], sbuf)

        @plsc.parallel_loop(0, 8, 1)
        def _(g):
            gb = g * 16
            cs = _splat(gb)
            for r in range(16):
                plsc.store_scatter(ov1, [b3v, cs + r], sbuf[gb + r])

        pltpu.sync_copy(ov1.at[pl.ds(0, 8), pl.ds(0, 128)],
                        out3.at[0, pl.ds(t * 8, 8)])
        pltpu.sync_copy(ov1.at[pl.ds(8, 8), pl.ds(0, 128)],
                        out3.at[1, pl.ds(t * 8, 8)])
        return carry

    lax.fori_loop(0, ncp, colfn, 0)

    nloc = (_NCH2 // _NW) + jnp.where(w < _NCH2 % _NW, 1, 0)

    def chunk_body(i, carry):
        cid = w + i * _NW
        base = cid * _C2
        cps = [pltpu.make_async_copy(w1.at[pl.ds(base, _C2)], w_v, sem)]
        cps += [
            pltpu.make_async_copy(heads1.at[pl.ds(base + j * 128, 128)],
                                  gidx_v.at[j], sem)
            for j in range(_NSUB2)
        ]
        for cp in cps:
            cp.start()
        for cp in cps:
            cp.wait()

        gs = [pltpu.make_async_copy(head_in.at[gidx_v.at[j]],
                                    val_v.at[pl.ds(j * 128, 128)], sem)
              for j in range(_NSUB2)]
        for g in gs:
            g.start()
        for g in gs:
            g.wait()

        @plsc.parallel_loop(0, _C2 // 16, 1)
        def _(g):
            gb = g * 16
            w16 = w_v[pl.ds(gb, 16)]
            row_v = b2v + (g // 8) * 8
            cs = _splat((g % 8) * 16)
            for r in range(16):
                row = val_v[gb + r] * w16[r]
                plsc.store_scatter(ov, [row_v, cs + r], row)
        mo = (_HDRTC + cid * _NSUB2) * 8
        nm = _NSUB2 * 8
        pltpu.sync_copy(ov.at[pl.ds(0, nm), pl.ds(0, 128)],
                        out3.at[0, pl.ds(mo, nm)])
        pltpu.sync_copy(ov.at[pl.ds(nm, nm), pl.ds(0, 128)],
                        out3.at[1, pl.ds(mo, nm)])
        return carry

    lax.fori_loop(0, nloc, chunk_body, 0)


def kernel(batch_heads, batch_rels, batch_tails, batch_ids, fact_ids,
           weight_list, fact_val):
    del fact_ids
    val3 = fact_val.reshape(_VTC, 128, 2, 8).transpose(2, 0, 3, 1).reshape(
        2, _NF // 16, 128)
    mesh = plsc.VectorSubcoreMesh(core_axis_name="c", subcore_axis_name="s")

    f32 = jnp.float32
    i32 = jnp.int32
    cparams = pltpu.CompilerParams(use_tc_tiling_on_sc=False,
                                   needs_layout_passes=False)
    p1 = pl.kernel(
        _p1_body,
        out_type=(
            jax.ShapeDtypeStruct((_NE, _D), f32),
            jax.ShapeDtypeStruct((_NE, _D), f32),
            jax.ShapeDtypeStruct((_NRB, _D), f32),
        ),
        mesh=mesh,
        scratch_types=[
            pltpu.VMEM_SHARED((_NE, _D), f32),
            pltpu.VMEM_SHARED((_NRB, _D), f32),
            pltpu.VMEM((2 * _NSUB1 * 8, 129), f32),
            pltpu.VMEM((_C1, _D), f32),
            pltpu.VMEM((_C1,), f32),
            pltpu.VMEM((_NSUB1, 128), i32),
            pltpu.VMEM((_C1,), i32),
            pltpu.VMEM((_C1,), i32),
            pltpu.VMEM((_NSUB1, 128), i32),
            pltpu.VMEM((2 * _NSUB1 * 8, 129), f32),
            pltpu.VMEM((_C1,), f32),
            pltpu.VMEM((_NSUB1, 128), i32),
            pltpu.SemaphoreType.DMA,
            pltpu.SemaphoreType.DMA,
        ],
        compiler_params=cparams,
    )
    head_agg, tail_agg, rel_agg = p1(batch_heads, batch_tails, batch_rels,
                                     batch_ids, weight_list, val3)

    p2 = pl.kernel(
        _p2_body,
        out_type=jax.ShapeDtypeStruct((2, _OTC * 8, 128), f32),
        mesh=mesh,
        scratch_types=[
            pltpu.VMEM((_C2, _D), f32),
            pltpu.VMEM((2 * _NSUB2 * 8, 129), f32),
            pltpu.VMEM((_C2,), f32),
            pltpu.VMEM((_NSUB2, 128), i32),
            pltpu.VMEM((128, _D), f32),
            pltpu.VMEM((16, 129), f32),
            pltpu.SemaphoreType.DMA,
        ],
        compiler_params=cparams,
    )
    out3 = p2(batch_heads, weight_list, head_agg, tail_agg, rel_agg)
    return out3.reshape(2, _OTC, 8, 128).transpose(1, 3, 0, 2).reshape(
        _OUT_ROWS, _D)


# speedup vs baseline: 5.0440x; 1.1500x over previous
import jax
import jax.numpy as jnp
from jax import lax
from jax.experimental import pallas as pl
from jax.experimental.pallas import tpu as pltpu, tpu_sc as plsc

_NE = 100_000
_NRB = 1_600
_NF = 3_200_000
_D = 16
_NREL = 200
_NC = 2
_NS = 16
_NW = _NC * _NS
_OUT_ROWS = 2 * _NE + _NRB + _NF
_OTC = _OUT_ROWS // 128
_VTC = _NF // 128
_HDRTC = (2 * _NE + _NRB) // 128
_HT_COL = _NE // 128
_TR_COL = 2 * _NE // 128

_C1 = 512
_NSUB1 = _C1 // 128
_NCH1 = _NF // _C1
_C2 = 1024
_NSUB2 = _C2 // 128
_NCH2 = _NF // _C2
_EZ = 400
_NEZ = _NE // _EZ


def _lane_consts(stride):
    iota = lax.iota(jnp.int32, 16)
    return (iota // 8) * stride + iota % 8, iota


def _splat(x):
    return jnp.broadcast_to(x, (16,))


def _p1_body(heads1, tails1, rels1, ids1, w1, val3,
             head_out, tail_out, rel_a_out, rel_b_out,
             acc_sh, rel_sh, vv0, val16_v, w_v0, sidx_v0, rels_v, ids_v,
             ridx_v, vv1, w_v1, sidx_v1, sem, sem2):
    c = lax.axis_index("c")
    s = lax.axis_index("s")
    b1v, _ = _lane_consts(_NSUB1 * 8)
    slots = ((vv0, w_v0, sidx_v0), (vv1, w_v1, sidx_v1))

    z = jnp.zeros((_D,), jnp.float32)

    def zb(r, carry):
        val16_v[r] = z
        return carry

    lax.fori_loop(0, _C1, zb, 0)
    nz = (_NEZ // _NS) + jnp.where(s < _NEZ % _NS, 1, 0)

    def zbody(i, carry):
        pltpu.sync_copy(val16_v.at[pl.ds(0, _EZ)],
                        acc_sh.at[pl.ds((s + i * _NS) * _EZ, _EZ)])
        return carry

    lax.fori_loop(0, nz, zbody, 0)

    @pl.when(s < _NRB // _EZ)
    def _():
        pltpu.sync_copy(val16_v.at[pl.ds(0, _EZ)],
                        rel_sh.at[pl.ds(s * _EZ, _EZ)])

    plsc.subcore_barrier()

    nloc = (_NCH1 // _NS) + jnp.where(s < _NCH1 % _NS, 1, 0)

    def run_core(is_head):
        idx_src = heads1 if is_head else tails1

        def stage_cps(slot, i):
            vv, w_v, sidx_v = slots[slot]
            cid = s + i * _NS
            base = cid * _C1
            tc0 = cid * _NSUB1
            cps = [
                pltpu.make_async_copy(val3.at[0, pl.ds(tc0 * 8, _NSUB1 * 8)],
                                      vv.at[pl.ds(0, _NSUB1 * 8),
                                            pl.ds(0, 128)], sem),
                pltpu.make_async_copy(val3.at[1, pl.ds(tc0 * 8, _NSUB1 * 8)],
                                      vv.at[pl.ds(_NSUB1 * 8, _NSUB1 * 8),
                                            pl.ds(0, 128)], sem),
                pltpu.make_async_copy(w1.at[pl.ds(base, _C1)], w_v, sem),
            ]
            cps += [
                pltpu.make_async_copy(
                    idx_src.at[pl.ds(base + j * 128, 128)],
                    sidx_v.at[j], sem)
                for j in range(_NSUB1)
            ]
            return cps

        def start_stage(slot, i):
            for cp in stage_cps(slot, i):
                cp.start()

        def compute_scatter(slot, i, do_rel):
            vv, w_v, sidx_v = slots[slot]
            if do_rel:
                base = (s + i * _NS) * _C1
                rcs = [pltpu.make_async_copy(rels1.at[pl.ds(base, _C1)],
                                             rels_v, sem2),
                       pltpu.make_async_copy(ids1.at[pl.ds(base, _C1)],
                                             ids_v, sem2)]
                for cp in rcs:
                    cp.start()
                for cp in rcs:
                    cp.wait()
            for cp in stage_cps(slot, i):
                cp.wait()

            @plsc.parallel_loop(0, _C1 // 16, 1)
            def _(g):
                gb = g * 16
                w16 = w_v[pl.ds(gb, 16)]
                if do_rel:
                    ridx_v[g // 8, pl.ds((g % 8) * 16, 16)] = (
                        rels_v[pl.ds(gb, 16)] + ids_v[pl.ds(gb, 16)] * _NREL)
                row_v = b1v + (g // 8) * 8
                cs = _splat((g % 8) * 16)
                for r in range(16):
                    row = plsc.load_gather(vv, [row_v, cs + r])
                    val16_v[gb + r] = row * w16[r]

            scs = [pltpu.make_async_copy(val16_v.at[pl.ds(j * 128, 128)],
                                         acc_sh.at[sidx_v.at[j]], sem2)
                   for j in range(_NSUB1)]
            if do_rel:
                scs += [pltpu.make_async_copy(
                    val16_v.at[pl.ds(j * 128, 128)],
                    rel_sh.at[ridx_v.at[j]], sem2)
                    for j in range(_NSUB1)]
            for sc_ in scs:
                sc_.start(add=True)
            for sc_ in scs:
                sc_.wait()

        start_stage(0, 0)

        def pair_body(h, carry):
            a = h * 2
            b = a + 1

            @pl.when(b < nloc)
            def _():
                start_stage(1, b)

            compute_scatter(0, a, is_head)

            @pl.when(a + 2 < nloc)
            def _():
                start_stage(0, a + 2)

            @pl.when(b < nloc)
            def _():
                compute_scatter(1, b, not is_head)

            return carry

        lax.fori_loop(0, (nloc + 1) // 2, pair_body, 0)

    @pl.when(c == 0)
    def _():
        run_core(True)

    @pl.when(c != 0)
    def _():
        run_core(False)

    plsc.subcore_barrier()

    def make_wb(dst):
        def wb(i, carry):
            b = (s + i * _NS) * _EZ
            pltpu.sync_copy(acc_sh.at[pl.ds(b, _EZ)], dst.at[pl.ds(b, _EZ)])
            return carry

        return wb

    @pl.when(c == 0)
    def _():
        lax.fori_loop(0, nz, make_wb(head_out), 0)

        @pl.when(s < _NRB // _EZ)
        def _():
            pltpu.sync_copy(rel_sh.at[pl.ds(s * _EZ, _EZ)],
                            rel_a_out.at[pl.ds(s * _EZ, _EZ)])

    @pl.when(c != 0)
    def _():
        lax.fori_loop(0, nz, make_wb(tail_out), 0)

        @pl.when(s < _NRB // _EZ)
        def _():
            pltpu.sync_copy(rel_sh.at[pl.ds(s * _EZ, _EZ)],
                            rel_b_out.at[pl.ds(s * _EZ, _EZ)])


def _p2_body(heads1, w1, head_in, tail_in, rel_a_in, rel_b_in, out3,
             val_v, ov, w_v, gidx_v, sbuf, sbuf2, ov1, sem):
    c = lax.axis_index("c")
    s = lax.axis_index("s")
    w = s * _NC + c
    b2v, iv = _lane_consts(_NSUB2 * 8)
    b3v, _ = _lane_consts(8)

    ncp = (_HDRTC // _NW) + jnp.where(w < _HDRTC % _NW, 1, 0)

    def colfn(i, carry):
        t = w + i * _NW
        rbase = t * 128

        @pl.when(t < _HT_COL)
        def _():
            pltpu.sync_copy(head_in.at[pl.ds(rbase, 128)], sbuf)

        @pl.when(t == _HT_COL)
        def _():
            pltpu.sync_copy(head_in.at[pl.ds(_HT_COL * 128, _NE % 128)],
                            sbuf.at[pl.ds(0, _NE % 128)])
            pltpu.sync_copy(tail_in.at[pl.ds(0, 128 - _NE % 128)],
                            sbuf.at[pl.ds(_NE % 128, 128 - _NE % 128)])

        @pl.when(jnp.logical_and(t > _HT_COL, t < _TR_COL))
        def _():
            pltpu.sync_copy(tail_in.at[pl.ds(rbase - _NE, 128)], sbuf)

        @pl.when(t == _TR_COL)
        def _():
            pltpu.sync_copy(tail_in.at[pl.ds(_TR_COL * 128 - _NE, 64)],
                            sbuf.at[pl.ds(0, 64)])
            pltpu.sync_copy(rel_a_in.at[pl.ds(0, 64)], sbuf.at[pl.ds(64, 64)])
            pltpu.sync_copy(rel_b_in.at[pl.ds(0, 64)], sbuf2.at[pl.ds(0, 64)])

            @plsc.parallel_loop(0, 4, 1)
            def _(g):
                for r in range(16):
                    f = g * 16 + r
                    sbuf[64 + f] = sbuf[64 + f] + sbuf2[f]

        @pl.when(t > _TR_COL)
        def _():
            pltpu.sync_copy(rel_a_in.at[pl.ds(rbase - 2 * _NE, 128)], sbuf)
            pltpu.sync_copy(rel_b_in.at[pl.ds(rbase - 2 * _NE, 128)], sbuf2)

            @plsc.parallel_loop(0, 8, 1)
            def _(g):
                for r in range(16):
                    f = g * 16 + r
                    sbuf[f] = sbuf[f] + sbuf2[f]

        @plsc.parallel_loop(0, 8, 1)
        def _(g):
            gb = g * 16
            cs = _splat(gb)
            for r in range(16):
                plsc.store_scatter(ov1, [b3v, cs + r], sbuf[gb + r])

        pltpu.sync_copy(ov1.at[pl.ds(0, 8), pl.ds(0, 128)],
                        out3.at[0, pl.ds(t * 8, 8)])
        pltpu.sync_copy(ov1.at[pl.ds(8, 8), pl.ds(0, 128)],
                        out3.at[1, pl.ds(t * 8, 8)])
        return carry

    lax.fori_loop(0, ncp, colfn, 0)

    nloc = (_NCH2 // _NW) + jnp.where(w < _NCH2 % _NW, 1, 0)

    def chunk_body(i, carry):
        cid = w + i * _NW
        base = cid * _C2
        cps = [pltpu.make_async_copy(w1.at[pl.ds(base, _C2)], w_v, sem)]
        cps += [
            pltpu.make_async_copy(heads1.at[pl.ds(base + j * 128, 128)],
                                  gidx_v.at[j], sem)
            for j in range(_NSUB2)
        ]
        for cp in cps:
            cp.start()
        for cp in cps:
            cp.wait()

        gs = [pltpu.make_async_copy(head_in.at[gidx_v.at[j]],
                                    val_v.at[pl.ds(j * 128, 128)], sem)
              for j in range(_NSUB2)]
        for g in gs:
            g.start()
        for g in gs:
            g.wait()

        @plsc.parallel_loop(0, _C2 // 16, 1)
        def _(g):
            gb = g * 16
            w16 = w_v[pl.ds(gb, 16)]
            row_v = b2v + (g // 8) * 8
            cs = _splat((g % 8) * 16)
            for r in range(16):
                row = val_v[gb + r] * w16[r]
                plsc.store_scatter(ov, [row_v, cs + r], row)
        mo = (_HDRTC + cid * _NSUB2) * 8
        nm = _NSUB2 * 8
        pltpu.sync_copy(ov.at[pl.ds(0, nm), pl.ds(0, 128)],
                        out3.at[0, pl.ds(mo, nm)])
        pltpu.sync_copy(ov.at[pl.ds(nm, nm), pl.ds(0, 128)],
                        out3.at[1, pl.ds(mo, nm)])
        return carry

    lax.fori_loop(0, nloc, chunk_body, 0)


def kernel(batch_heads, batch_rels, batch_tails, batch_ids, fact_ids,
           weight_list, fact_val):
    del fact_ids
    val3 = fact_val.reshape(_VTC, 128, 2, 8).transpose(2, 0, 3, 1).reshape(
        2, _NF // 16, 128)
    mesh = plsc.VectorSubcoreMesh(core_axis_name="c", subcore_axis_name="s")

    f32 = jnp.float32
    i32 = jnp.int32
    cparams = pltpu.CompilerParams(use_tc_tiling_on_sc=False,
                                   needs_layout_passes=False)
    p1 = pl.kernel(
        _p1_body,
        out_type=(
            jax.ShapeDtypeStruct((_NE, _D), f32),
            jax.ShapeDtypeStruct((_NE, _D), f32),
            jax.ShapeDtypeStruct((_NRB, _D), f32),
            jax.ShapeDtypeStruct((_NRB, _D), f32),
        ),
        mesh=mesh,
        scratch_types=[
            pltpu.VMEM_SHARED((_NE, _D), f32),
            pltpu.VMEM_SHARED((_NRB, _D), f32),
            pltpu.VMEM((2 * _NSUB1 * 8, 129), f32),
            pltpu.VMEM((_C1, _D), f32),
            pltpu.VMEM((_C1,), f32),
            pltpu.VMEM((_NSUB1, 128), i32),
            pltpu.VMEM((_C1,), i32),
            pltpu.VMEM((_C1,), i32),
            pltpu.VMEM((_NSUB1, 128), i32),
            pltpu.VMEM((2 * _NSUB1 * 8, 129), f32),
            pltpu.VMEM((_C1,), f32),
            pltpu.VMEM((_NSUB1, 128), i32),
            pltpu.SemaphoreType.DMA,
            pltpu.SemaphoreType.DMA,
        ],
        compiler_params=cparams,
    )
    head_agg, tail_agg, rel_a, rel_b = p1(batch_heads, batch_tails,
                                          batch_rels, batch_ids,
                                          weight_list, val3)

    p2 = pl.kernel(
        _p2_body,
        out_type=jax.ShapeDtypeStruct((2, _OTC * 8, 128), f32),
        mesh=mesh,
        scratch_types=[
            pltpu.VMEM((_C2, _D), f32),
            pltpu.VMEM((2 * _NSUB2 * 8, 129), f32),
            pltpu.VMEM((_C2,), f32),
            pltpu.VMEM((_NSUB2, 128), i32),
            pltpu.VMEM((128, _D), f32),
            pltpu.VMEM((128, _D), f32),
            pltpu.VMEM((16, 129), f32),
            pltpu.SemaphoreType.DMA,
        ],
        compiler_params=cparams,
    )
    out3 = p2(batch_heads, weight_list, head_agg, tail_agg, rel_a, rel_b)
    return out3.reshape(2, _OTC, 8, 128).transpose(1, 3, 0, 2).reshape(
        _OUT_ROWS, _D)
